# baseline (speedup 1.0000x reference)
import functools

import jax
import jax.numpy as jnp
from jax import lax
from jax.experimental import pallas as pl
from jax.experimental.pallas import tpu as pltpu
from jax.experimental.pallas import tpu_sc as plsc

_ROWS = 8192
_DIM = 1024

_info = plsc.get_sparse_core_info()
_NC = _info.num_cores
_NS = _info.num_subcores
_NW = _NC * _NS
_RPW = _ROWS // _NW


def _make_sc_copy():
    mesh = plsc.VectorSubcoreMesh(core_axis_name="c", subcore_axis_name="s")

    @functools.partial(
        pl.kernel,
        mesh=mesh,
        out_type=jax.ShapeDtypeStruct((_ROWS, _DIM), jnp.float32),
    )
    def sc_copy(table_hbm, out_hbm):
        wid = lax.axis_index("s") * _NC + lax.axis_index("c")
        base = wid * _RPW
        pltpu.sync_copy(
            table_hbm.at[pl.ds(base, _RPW)],
            out_hbm.at[pl.ds(base, _RPW)],
        )

    return sc_copy


_sc_copy = _make_sc_copy()


@jax.jit
def kernel(x, pos_emb):
    del x
    return _sc_copy(pos_emb)


# speedup vs baseline: 41.9246x; 41.9246x over previous
import jax
import jax.numpy as jnp
from jax.experimental import pallas as pl
from jax.experimental.pallas import tpu as pltpu

_ROWS = 8192
_DIM = 1024
_BLK = 512


def _copy_body(in_ref, out_ref):
    out_ref[...] = in_ref[...]


_tc_copy = pl.pallas_call(
    _copy_body,
    grid=(_ROWS // _BLK,),
    in_specs=[pl.BlockSpec((_BLK, _DIM), lambda i: (i, 0))],
    out_specs=pl.BlockSpec((_BLK, _DIM), lambda i: (i, 0)),
    out_shape=jax.ShapeDtypeStruct((_ROWS, _DIM), jnp.float32),
)


@jax.jit
def kernel(x, pos_emb):
    del x
    return _tc_copy(pos_emb)
